# baseline (speedup 1.0000x reference)
import functools

import jax
import jax.numpy as jnp
from jax import lax
from jax.experimental import pallas as pl
from jax.experimental.pallas import tpu as pltpu
from jax.experimental.pallas import tpu_sc as plsc

B, N, C, M = 8, 65536, 128, 16384
NW = 32
WPB = NW // B
CPW = C // WPB
MPW = M // WPB
XC = N


def _sc_body(xyzf_hbm, feat_hbm, inds_hbm,
             out_xyz_hbm, out_feat_hbm,
             idx_v, row_v, fout_v, xout_v):
    cid = lax.axis_index("c")
    sid = lax.axis_index("s")
    wid = sid * 2 + cid
    b = wid // WPB
    q = wid % WPB
    c0 = q * CPW

    pltpu.sync_copy(inds_hbm.at[b], idx_v)

    for p in range(3):
        pltpu.sync_copy(xyzf_hbm.at[b, pl.ds(p * XC, XC)], row_v)

        def _xpass(i, carry, p=p):
            base = pl.multiple_of(i * 16, 16)
            iv = idx_v[pl.ds(q * MPW + base, 16)]
            e0 = iv * 3 - p * XC
            tgt0 = (base + lax.iota(jnp.int32, 16)) * 3
            for j in range(3):
                ej = e0 + j
                mask = (ej >= 0) & (ej < XC)
                ec = jnp.clip(ej, 0, XC - 1)
                vals = plsc.load_gather(row_v, [ec], mask=mask)
                plsc.store_scatter(xout_v, [tgt0 + j], vals, mask=mask)
            return carry

        lax.fori_loop(0, MPW // 16, _xpass, 0)

    pltpu.sync_copy(xout_v, out_xyz_hbm.at[b, pl.ds(q * (3 * MPW), 3 * MPW)])

    def _chan(j, carry):
        c = c0 + j
        pltpu.sync_copy(feat_hbm.at[b, c], row_v)

        def _gather(i, carry2):
            base = pl.multiple_of(i * 128, 128)
            for u in range(8):
                s = pl.ds(base + u * 16, 16)
                iv = idx_v[s]
                fout_v[s] = plsc.load_gather(row_v, [iv])
            return carry2

        lax.fori_loop(0, M // 128, _gather, 0)
        pltpu.sync_copy(fout_v, out_feat_hbm.at[b, c])
        return carry

    lax.fori_loop(0, CPW, _chan, 0)


@jax.jit
def _sc_gather(xyzf, features, inds):
    mesh = plsc.VectorSubcoreMesh(core_axis_name="c", subcore_axis_name="s")
    kern = functools.partial(
        pl.kernel,
        mesh=mesh,
        compiler_params=pltpu.CompilerParams(needs_layout_passes=False),
        out_type=(
            jax.ShapeDtypeStruct((B, 3 * M), jnp.float32),
            jax.ShapeDtypeStruct((B, C, M), jnp.float32),
        ),
        scratch_types=[
            pltpu.VMEM((M,), jnp.int32),
            pltpu.VMEM((N,), jnp.float32),
            pltpu.VMEM((M,), jnp.float32),
            pltpu.VMEM((3 * MPW,), jnp.float32),
        ],
    )(_sc_body)
    return kern(xyzf, features, inds)


def kernel(xyz, features, sample_inds):
    inds = sample_inds.astype(jnp.int32)
    xyzf = xyz.reshape(B, 3 * N)
    out_xyz, out_feat = _sc_gather(xyzf, features, inds)
    return (out_xyz.reshape(B, M, 3), out_feat, sample_inds)
